# baseline (speedup 1.0000x reference)
import functools

import jax
import jax.numpy as jnp
from jax import lax
from jax.experimental import pallas as pl
from jax.experimental.pallas import tpu as pltpu
from jax.experimental.pallas import tpu_sc as plsc

BLK = 1024
SC_BLOCKS = 2
SLAB = 8


def _matvec_kernel(x_ref, w_ref, b_ref, out_ref, *, blk):
    xb = x_ref[...].astype(jnp.bfloat16)
    wb = w_ref[...].astype(jnp.bfloat16)
    logits = jnp.dot(xb, wb, preferred_element_type=jnp.float32)
    probs = jax.nn.sigmoid(logits + b_ref[0, 0])
    out_ref[...] = probs.reshape(1, 1, blk)


def _lane_perm(v, idx):
    return lax.gather(
        v, idx[:, None],
        lax.GatherDimensionNumbers(
            offset_dims=(), collapsed_slice_dims=(0,), start_index_map=(0,)),
        slice_sizes=(1,),
        mode=lax.GatherScatterMode.PROMISE_IN_BOUNDS)


def _rne_bf16(v):
    bits = lax.bitcast_convert_type(v, jnp.int32)
    odd = lax.shift_right_logical(bits, 16) & 1
    r = bits + 0x7FFF + odd
    return lax.bitcast_convert_type(r & jnp.int32(-65536), jnp.float32)


def _sc_probs_kernel(x_hbm, w_hbm, b_hbm, out_hbm,
                     w_v, b_v, xs0, xs1, out_v, sem0, sem1,
                     *, base0, rows, slab, h):
    c = lax.axis_index("c")
    s = lax.axis_index("s")
    wid = s * 2 + c
    base = base0 + wid * rows

    pltpu.sync_copy(w_hbm, w_v)
    pltpu.sync_copy(b_hbm, b_v)

    def _round_w(j, _):
        w_v[pl.ds(j * 16, 16)] = _rne_bf16(w_v[pl.ds(j * 16, 16)])
        return 0
    lax.fori_loop(0, h // 16, _round_w, 0)

    bufs = (xs0, xs1)
    sems = (sem0, sem1)
    nslab = rows // slab
    lane = jnp.arange(16, dtype=jnp.int32)
    perms = [lane ^ sh for sh in (8, 4, 2, 1)]
    handles = [pltpu.async_copy(x_hbm.at[pl.ds(base, slab)], xs0, sem0)]
    rowvec = jnp.zeros((16,), jnp.float32)
    for g in range(nslab):
        handles[g].wait()
        if g + 1 < nslab:
            handles.append(pltpu.async_copy(
                x_hbm.at[pl.ds(base + (g + 1) * slab, slab)],
                bufs[(g + 1) % 2], sems[(g + 1) % 2]))
        buf = bufs[g % 2]

        def _row(r, rv, g=g, buf=buf):
            def _chunk(j, acc):
                xc = _rne_bf16(buf[r, pl.ds(j * 16, 16)])
                return acc + xc * w_v[pl.ds(j * 16, 16)]
            acc = lax.fori_loop(0, h // 16, _chunk,
                                jnp.zeros((16,), jnp.float32), unroll=16)
            tot = acc
            for perm in perms:
                tot = tot + _lane_perm(tot, perm)
            pos = (g % (16 // slab)) * slab + r
            return jnp.where(lane == pos, tot, rv)
        rowvec = lax.fori_loop(0, slab, _row, rowvec)
        if g % (16 // slab) == (16 // slab) - 1:
            out_v[pl.ds((g // (16 // slab)) * 16, 16)] = rowvec
            rowvec = jnp.zeros((16,), jnp.float32)

    def _sig(j, _):
        z = out_v[pl.ds(j * 16, 16)] + b_v[...]
        out_v[pl.ds(j * 16, 16)] = 1.0 / (1.0 + jnp.exp(-z))
        return 0
    lax.fori_loop(0, rows // 16, _sig, 0)
    pltpu.sync_copy(out_v, out_hbm.at[pl.ds(wid * rows, rows)])


def _select_kernel(ptc_ref, psc_ref, wtc_ref, wsc_ref, *, k):
    bits_tc = ptc_ref[...].view(jnp.int32)
    bits_sc = psc_ref[...].view(jnp.int32)

    def body(_, lohi):
        lo, hi = lohi
        mid = lax.div(lo + hi, 2)
        cnt = (jnp.sum((bits_tc <= mid).astype(jnp.int32))
               + jnp.sum((bits_sc <= mid).astype(jnp.int32)))
        return jnp.where(cnt >= k, lo, mid + 1), jnp.where(cnt >= k, mid, hi)

    lo = jnp.int32(0)
    hi = jnp.int32(0x3F800000)
    lo, hi = lax.fori_loop(0, 31, body, (lo, hi))
    wtc_ref[...] = (bits_tc >= lo).astype(jnp.float32)
    wsc_ref[...] = (bits_sc >= lo).astype(jnp.float32)


def kernel(x, W, b):
    B, S, H = x.shape
    total = B * S
    capacity = int(total * 0.5)
    k = max(1, total - capacity)

    NB = total // BLK
    NB_TC = NB - SC_BLOCKS
    TC_ROWS = NB_TC * BLK
    SC_ROWS = SC_BLOCKS * BLK
    R = SC_ROWS // 32

    xf = x.reshape(total, H)
    wt = W.reshape(H, 1)
    w1 = W.reshape(H)
    b2 = b.reshape(1, 1)
    b16 = jnp.broadcast_to(b.reshape(1), (16,))

    sc_call = pl.kernel(
        functools.partial(_sc_probs_kernel, base0=TC_ROWS, rows=R,
                          slab=SLAB, h=H),
        out_type=jax.ShapeDtypeStruct((SC_ROWS,), jnp.float32),
        mesh=plsc.VectorSubcoreMesh(core_axis_name="c", subcore_axis_name="s"),
        scratch_types=[
            pltpu.VMEM((H,), jnp.float32),
            pltpu.VMEM((16,), jnp.float32),
            pltpu.VMEM((SLAB, H), jnp.float32),
            pltpu.VMEM((SLAB, H), jnp.float32),
            pltpu.VMEM((R,), jnp.float32),
            pltpu.SemaphoreType.DMA,
            pltpu.SemaphoreType.DMA,
        ],
    )
    probs_sc = sc_call(xf, w1, b16)

    half_nb = NB_TC // 2
    tc_parts = []
    for p in range(2):
        nblk = half_nb if p == 0 else NB_TC - half_nb
        off = p * half_nb
        tc_parts.append(pl.pallas_call(
            functools.partial(_matvec_kernel, blk=BLK),
            grid=(nblk,),
            in_specs=[
                pl.BlockSpec((BLK, H), lambda i, off=off: (i + off, 0)),
                pl.BlockSpec((H, 1), lambda i: (0, 0)),
                pl.BlockSpec((1, 1), lambda i: (0, 0)),
            ],
            out_specs=pl.BlockSpec((1, 1, BLK), lambda i: (i, 0, 0)),
            out_shape=jax.ShapeDtypeStruct((nblk, 1, BLK), jnp.float32),
        )(xf, wt, b2).reshape(nblk, BLK))
    probs_tc = jnp.concatenate(tc_parts, axis=0)

    psc2 = probs_sc.reshape(SC_BLOCKS, BLK)
    wtc, wsc = pl.pallas_call(
        functools.partial(_select_kernel, k=k),
        out_shape=[
            jax.ShapeDtypeStruct((NB_TC, BLK), jnp.float32),
            jax.ShapeDtypeStruct((SC_BLOCKS, BLK), jnp.float32),
        ],
    )(probs_tc, psc2)

    weights = jnp.concatenate([wtc.reshape(-1), wsc.reshape(-1)])
    probs = jnp.concatenate([probs_tc.reshape(-1), probs_sc])
    return (weights.reshape(B, S, 1), probs.reshape(B, S, 1))


# speedup vs baseline: 1.0357x; 1.0357x over previous
import functools

import jax
import jax.numpy as jnp
from jax import lax
from jax.experimental import pallas as pl
from jax.experimental.pallas import tpu as pltpu
from jax.experimental.pallas import tpu_sc as plsc

BLK = 1024
SC_BLOCKS = 2
SLAB = 8


def _matvec_kernel(x_ref, w_ref, b_ref, out_ref, *, blk):
    xb = x_ref[...].astype(jnp.bfloat16)
    wb = w_ref[...].astype(jnp.bfloat16)
    logits = jnp.dot(xb, wb, preferred_element_type=jnp.float32)
    probs = jax.nn.sigmoid(logits + b_ref[0, 0])
    out_ref[...] = probs.reshape(1, 1, blk)


def _lane_perm(v, idx):
    return lax.gather(
        v, idx[:, None],
        lax.GatherDimensionNumbers(
            offset_dims=(), collapsed_slice_dims=(0,), start_index_map=(0,)),
        slice_sizes=(1,),
        mode=lax.GatherScatterMode.PROMISE_IN_BOUNDS)


def _rne_bf16(v):
    bits = lax.bitcast_convert_type(v, jnp.int32)
    odd = lax.shift_right_logical(bits, 16) & 1
    r = bits + 0x7FFF + odd
    return lax.bitcast_convert_type(r & jnp.int32(-65536), jnp.float32)


def _sc_probs_kernel(x_hbm, w_hbm, b_hbm, out_hbm,
                     w_v, b_v, xs0, xs1, out_v, sem0, sem1,
                     *, base0, rows, slab, h):
    c = lax.axis_index("c")
    s = lax.axis_index("s")
    wid = s * 2 + c
    base = base0 + wid * rows

    pltpu.sync_copy(w_hbm, w_v)
    pltpu.sync_copy(b_hbm, b_v)

    def _round_w(j, _):
        w_v[pl.ds(j * 16, 16)] = _rne_bf16(w_v[pl.ds(j * 16, 16)])
        return 0
    lax.fori_loop(0, h // 16, _round_w, 0)

    bufs = (xs0, xs1)
    sems = (sem0, sem1)
    nslab = rows // slab
    lane = jnp.arange(16, dtype=jnp.int32)
    perms = [lane ^ sh for sh in (8, 4, 2, 1)]
    handles = [pltpu.async_copy(x_hbm.at[pl.ds(base, slab)], xs0, sem0)]
    rowvec = jnp.zeros((16,), jnp.float32)
    for g in range(nslab):
        handles[g].wait()
        if g + 1 < nslab:
            handles.append(pltpu.async_copy(
                x_hbm.at[pl.ds(base + (g + 1) * slab, slab)],
                bufs[(g + 1) % 2], sems[(g + 1) % 2]))
        buf = bufs[g % 2]

        def _row(r, rv, g=g, buf=buf):
            def _chunk(j, acc):
                xc = _rne_bf16(buf[r, pl.ds(j * 16, 16)])
                return acc + xc * w_v[pl.ds(j * 16, 16)]
            acc = lax.fori_loop(0, h // 16, _chunk,
                                jnp.zeros((16,), jnp.float32), unroll=16)
            tot = acc
            for perm in perms:
                tot = tot + _lane_perm(tot, perm)
            pos = (g % (16 // slab)) * slab + r
            return jnp.where(lane == pos, tot, rv)
        rowvec = lax.fori_loop(0, slab, _row, rowvec)
        if g % (16 // slab) == (16 // slab) - 1:
            out_v[pl.ds((g // (16 // slab)) * 16, 16)] = rowvec
            rowvec = jnp.zeros((16,), jnp.float32)

    def _sig(j, _):
        z = out_v[pl.ds(j * 16, 16)] + b_v[...]
        out_v[pl.ds(j * 16, 16)] = 1.0 / (1.0 + jnp.exp(-z))
        return 0
    lax.fori_loop(0, rows // 16, _sig, 0)
    pltpu.sync_copy(out_v, out_hbm.at[pl.ds(wid * rows, rows)])


def _select_kernel(ptc_ref, psc_ref, wtc_ref, wsc_ref, *, k):
    bits_tc = ptc_ref[...].view(jnp.int32)
    bits_sc = psc_ref[...].view(jnp.int32)

    def body(_, lohi):
        lo, hi = lohi
        mid = lax.div(lo + hi, 2)
        cnt = (jnp.sum((bits_tc <= mid).astype(jnp.int32))
               + jnp.sum((bits_sc <= mid).astype(jnp.int32)))
        return jnp.where(cnt >= k, lo, mid + 1), jnp.where(cnt >= k, mid, hi)

    lo = jnp.int32(0)
    hi = jnp.int32(0x3F800000)
    lo, hi = lax.fori_loop(0, 31, body, (lo, hi))
    wtc_ref[...] = (bits_tc >= lo).astype(jnp.float32)
    wsc_ref[...] = (bits_sc >= lo).astype(jnp.float32)


def kernel(x, W, b):
    B, S, H = x.shape
    total = B * S
    capacity = int(total * 0.5)
    k = max(1, total - capacity)

    NB = total // BLK
    NB_TC = NB - SC_BLOCKS
    TC_ROWS = NB_TC * BLK
    SC_ROWS = SC_BLOCKS * BLK
    R = SC_ROWS // 32

    xf = x.reshape(total, H)
    wt = W.reshape(H, 1)
    w1 = W.reshape(H)
    b2 = b.reshape(1, 1)
    b16 = jnp.broadcast_to(b.reshape(1), (16,))

    sc_call = pl.kernel(
        functools.partial(_sc_probs_kernel, base0=TC_ROWS, rows=R,
                          slab=SLAB, h=H),
        out_type=jax.ShapeDtypeStruct((SC_ROWS,), jnp.float32),
        mesh=plsc.VectorSubcoreMesh(core_axis_name="c", subcore_axis_name="s"),
        scratch_types=[
            pltpu.VMEM((H,), jnp.float32),
            pltpu.VMEM((16,), jnp.float32),
            pltpu.VMEM((SLAB, H), jnp.float32),
            pltpu.VMEM((SLAB, H), jnp.float32),
            pltpu.VMEM((R,), jnp.float32),
            pltpu.SemaphoreType.DMA,
            pltpu.SemaphoreType.DMA,
        ],
    )
    probs_tc = pl.pallas_call(
        functools.partial(_matvec_kernel, blk=BLK),
        grid=(NB_TC,),
        in_specs=[
            pl.BlockSpec((BLK, H), lambda i: (i, 0)),
            pl.BlockSpec((H, 1), lambda i: (0, 0)),
            pl.BlockSpec((1, 1), lambda i: (0, 0)),
        ],
        out_specs=pl.BlockSpec((1, 1, BLK), lambda i: (i, 0, 0)),
        out_shape=jax.ShapeDtypeStruct((NB_TC, 1, BLK), jnp.float32),
    )(xf, wt, b2).reshape(NB_TC, BLK)

    probs_sc = sc_call(xf, w1, b16)

    psc2 = probs_sc.reshape(SC_BLOCKS, BLK)
    wtc, wsc = pl.pallas_call(
        functools.partial(_select_kernel, k=k),
        out_shape=[
            jax.ShapeDtypeStruct((NB_TC, BLK), jnp.float32),
            jax.ShapeDtypeStruct((SC_BLOCKS, BLK), jnp.float32),
        ],
    )(probs_tc, psc2)

    weights = jnp.concatenate([wtc.reshape(-1), wsc.reshape(-1)])
    probs = jnp.concatenate([probs_tc.reshape(-1), probs_sc])
    return (weights.reshape(B, S, 1), probs.reshape(B, S, 1))


# speedup vs baseline: 1.1598x; 1.1198x over previous
import functools

import jax
import jax.numpy as jnp
from jax.experimental import pallas as pl


def _router_kernel(xa_ref, xb_ref, w_ref, b_ref, probs_ref, weights_ref,
                   *, k, nb, blk, h):
    i = pl.program_id(0)

    wb = w_ref[...].astype(jnp.bfloat16)
    half = blk // 2
    row = []
    for x_ref in (xa_ref, xb_ref):
        xb16 = x_ref[...].astype(jnp.bfloat16)
        logits = jnp.dot(xb16, wb, preferred_element_type=jnp.float32)
        probs = jax.nn.sigmoid(logits + b_ref[0, 0])
        row.append(probs.reshape(1, half))
    probs_ref[pl.ds(i, 1), :] = jnp.concatenate(row, axis=1)

    @pl.when(i == nb - 1)
    def _():
        bits = probs_ref[...].view(jnp.int32)

        def body(_, lohi):
            lo, hi = lohi
            mid = jax.lax.div(lo + hi, 2)
            cnt = jnp.sum((bits <= mid).astype(jnp.int32))
            return jnp.where(cnt >= k, lo, mid + 1), jnp.where(cnt >= k, mid, hi)

        lo = jnp.int32(0)
        hi = jnp.int32(0x3F800000)
        lo, hi = jax.lax.fori_loop(0, 31, body, (lo, hi))
        weights_ref[...] = (bits >= lo).astype(jnp.float32)


def kernel(x, W, b):
    B, S, H = x.shape
    total = B * S
    capacity = int(total * 0.5)
    k = max(1, total - capacity)

    BLK = 1024
    NB = total // BLK
    HALF = BLK // 2
    xf = x.reshape(total, H)
    wt = W.reshape(H, 1)
    b2 = b.reshape(1, 1)

    probs, weights = pl.pallas_call(
        functools.partial(_router_kernel, k=k, nb=NB, blk=BLK, h=H),
        grid=(NB,),
        in_specs=[
            pl.BlockSpec((HALF, H), lambda i: (2 * i, 0)),
            pl.BlockSpec((HALF, H), lambda i: (2 * i + 1, 0)),
            pl.BlockSpec((H, 1), lambda i: (0, 0)),
            pl.BlockSpec((1, 1), lambda i: (0, 0)),
        ],
        out_specs=[
            pl.BlockSpec((NB, BLK), lambda i: (0, 0)),
            pl.BlockSpec((NB, BLK), lambda i: (0, 0)),
        ],
        out_shape=[
            jax.ShapeDtypeStruct((NB, BLK), jnp.float32),
            jax.ShapeDtypeStruct((NB, BLK), jnp.float32),
        ],
    )(xf, xf, wt, b2)

    return (weights.reshape(B, S, 1), probs.reshape(B, S, 1))
